# Initial kernel scaffold; baseline (speedup 1.0000x reference)
import jax
import jax.numpy as jnp
from jax.experimental import pallas as pl


def kernel(skeleton, video_tokens, pos_tokens, num_tubes, spatial_per_tube, tubelet_size):
    raise NotImplementedError("write your pallas kernel here")


# speedup vs baseline: 1.8544x; 1.8544x over previous
import functools

import jax
import jax.numpy as jnp
from jax import lax
from jax.experimental import pallas as pl
from jax.experimental.pallas import tpu as pltpu
from jax.experimental.pallas import tpu_sc as plsc

_B = 16
_N = 3136
_D = 768
_T = 16
_S = 196
_KT = 8
_KN = _KT * _S

_NW = 32
_ROWS = _B * _KN
_RPW = _ROWS // _NW
_CHUNK = 56
_NCHUNK = _RPW // _CHUNK


def _saliency_rank_body(sk_ref, keep_ref, gidx_ref):
    sk = sk_ref[...]
    vel = sk[:, 1:] - sk[:, :-1]
    speed = jnp.sqrt(jnp.sum(vel * vel, axis=-1))
    spd = jnp.mean(speed, axis=-1)
    fs = jnp.concatenate([spd[:, :1], spd], axis=1)
    fs = fs / (jnp.max(fs, axis=1, keepdims=True) + 1e-6)
    ts = jnp.mean(fs.reshape(_B, _T, 2), axis=-1)

    tj = ts[:, None, :]
    tt = ts[:, :, None]
    jj = lax.broadcasted_iota(jnp.int32, (_B, _T, _T), 2)
    ii = lax.broadcasted_iota(jnp.int32, (_B, _T, _T), 1)
    before = (tj > tt) | ((tj == tt) & (jj < ii))
    rank = jnp.sum(before.astype(jnp.int32), axis=2)

    rr = lax.broadcasted_iota(jnp.int32, (_B, _T, _KT), 2)
    t3 = lax.broadcasted_iota(jnp.int32, (_B, _T, _KT), 1)
    onehot = rank[:, :, None] == rr
    order = jnp.sum(jnp.where(onehot, t3, 0), axis=1)

    ss = lax.broadcasted_iota(jnp.int32, (_B, _KT, _S), 2)
    ki = (order[:, :, None] * _S + ss).reshape(_B, _KN)
    keep_ref[...] = ki
    bb = lax.broadcasted_iota(jnp.int32, (_B, _KN), 0)
    gidx_ref[...] = ki + bb * _N


def _saliency_rank(skeleton):
    return pl.pallas_call(
        _saliency_rank_body,
        out_shape=[jax.ShapeDtypeStruct((_B, _KN), jnp.int32)] * 2,
    )(skeleton)


def _build_sc_gather():
    mesh = plsc.VectorSubcoreMesh(core_axis_name="c", subcore_axis_name="s")

    @functools.partial(
        pl.kernel,
        mesh=mesh,
        out_type=[jax.ShapeDtypeStruct((_ROWS, _D), jnp.float32)] * 2,
        scratch_types=[
            pltpu.VMEM((_RPW,), jnp.int32),
            pltpu.VMEM((_CHUNK, _D), jnp.float32),
            pltpu.VMEM((_CHUNK, _D), jnp.float32),
            pltpu.SemaphoreType.DMA,
            pltpu.SemaphoreType.DMA,
            pltpu.SemaphoreType.DMA,
            pltpu.SemaphoreType.DMA,
        ],
    )
    def gather_k(v_hbm, p_hbm, gidx_hbm, outv_hbm, outp_hbm,
                 idx_v, buf0, buf1, g0, g1, w0, w1):
        wid = lax.axis_index("s") * 2 + lax.axis_index("c")
        base = wid * _RPW
        pltpu.sync_copy(gidx_hbm.at[pl.ds(base, _RPW)], idx_v)

        bufs = (buf0, buf1)
        gsems = (g0, g1)
        wsems = (w0, w1)
        units = []
        for src, dst in ((v_hbm, outv_hbm), (p_hbm, outp_hbm)):
            for c in range(_NCHUNK):
                units.append((src, dst, c * _CHUNK))
        n = len(units)
        gd = [None] * n
        wd = [None] * n

        def start_gather(i):
            src, _, off = units[i]
            gd[i] = pltpu.async_copy(
                src.at[idx_v.at[pl.ds(off, _CHUNK)]], bufs[i % 2], gsems[i % 2])

        def start_write(i):
            _, dst, off = units[i]
            wd[i] = pltpu.async_copy(
                bufs[i % 2], dst.at[pl.ds(base + off, _CHUNK)], wsems[i % 2])

        start_gather(0)
        for i in range(n):
            if i + 1 < n:
                if i >= 1:
                    wd[i - 1].wait()
                start_gather(i + 1)
            gd[i].wait()
            start_write(i)
        wd[n - 2].wait()
        wd[n - 1].wait()

    return gather_k


def kernel(skeleton, video_tokens, pos_tokens, num_tubes, spatial_per_tube,
           tubelet_size):
    del num_tubes, spatial_per_tube, tubelet_size
    keep_idx, gidx = _saliency_rank(skeleton)
    vrows = video_tokens.reshape(_B * _N, _D)
    prows = pos_tokens.reshape(_B * _N, _D)
    outv, outp = _build_sc_gather()(vrows, prows, gidx.reshape(-1))
    return (outv.reshape(_B, _KN, _D), outp.reshape(_B, _KN, _D), keep_idx)
